# Initial kernel scaffold; baseline (speedup 1.0000x reference)
import jax
import jax.numpy as jnp
from jax.experimental import pallas as pl


def kernel(positive_edges, negative_edges, target, X, Wp1, bp1, Wn1, bn1, Wp2, bp2, Wn2, bn2, Wd, bd, none_edges, k_pos, k_neg):
    raise NotImplementedError("write your pallas kernel here")


# speedup vs baseline: 2.6212x; 2.6212x over previous
import functools

import jax
import jax.numpy as jnp
from jax import lax
from jax.experimental import pallas as pl
from jax.experimental.pallas import tpu as pltpu
from jax.experimental.pallas import tpu_sc as plsc

N = 10000
E = 80000
D = 256
H = 64
H2 = 32
LAMB = 5.0

NC, NS, L = 2, 16, 16
NW = NC * NS
N16 = 10240
RPS = N16 // NS
CH = 128
W1 = 48
PT2 = 5120
PT6 = 2560
PT6N = 5120

_mesh = plsc.VectorSubcoreMesh(core_axis_name="c", subcore_axis_name="s",
                               num_cores=NC, num_subcores=NS)


def _pad_idx(a, per, pad, padval):
    a = a.reshape(-1, per)
    return jnp.pad(a, ((0, 0), (0, pad - per)), constant_values=padval).reshape(-1)


def _k1_body(x_ref, w_ref, b_ref, t1_ref, xd_ref):
    y = jnp.dot(x_ref[...], w_ref[...], preferred_element_type=jnp.float32)
    ones = jnp.ones((N, 1), jnp.float32)
    zpad = jnp.zeros((N, W1 - 33), jnp.float32)
    t1_ref[0:N, :] = jnp.concatenate([y[:, 0:32], ones, zpad], axis=1)
    t1_ref[N16:N16 + N, :] = jnp.concatenate([y[:, 32:64], ones, zpad], axis=1)
    xd_ref[...] = y[:, 64:128] + b_ref[...]


def _k1(x, w1cat, b1cat):
    return pl.pallas_call(
        _k1_body,
        out_shape=[jax.ShapeDtypeStruct((2 * N16, W1), jnp.float32),
                   jax.ShapeDtypeStruct((N, H), jnp.float32)],
    )(x, w1cat, b1cat)


def _make_segsum(width):
    nch = PT2 // CH

    @functools.partial(
        pl.kernel, mesh=_mesh,
        out_type=jax.ShapeDtypeStruct((2 * N16, width), jnp.float32),
        compiler_params=pltpu.CompilerParams(use_tc_tiling_on_sc=False, needs_layout_passes=False),
        scratch_types=[
            pltpu.VMEM((CH,), jnp.int32),
            pltpu.VMEM((CH,), jnp.int32),
            pltpu.VMEM((CH, width), jnp.float32),
            pltpu.VMEM_SHARED((N16, width), jnp.float32),
            pltpu.SemaphoreType.DMA,
        ],
    )
    def k(table, srcp, dstp, zeros, out, src_v, dst_v, rows_v, acc, sem):
        c = lax.axis_index("c")
        s = lax.axis_index("s")
        r0 = s * RPS
        pltpu.sync_copy(zeros.at[pl.ds(r0, RPS)], acc.at[pl.ds(r0, RPS)])
        plsc.subcore_barrier()
        base = (c * NS + s) * PT2

        def body(ch, carry):
            off = base + ch * CH
            pltpu.sync_copy(srcp.at[pl.ds(off, CH)], src_v)
            pltpu.sync_copy(dstp.at[pl.ds(off, CH)], dst_v)
            pltpu.async_copy(table.at[src_v], rows_v, sem).wait()
            pltpu.sync_copy(rows_v, acc.at[dst_v], add=True)
            return carry

        lax.fori_loop(0, nch, body, 0)
        plsc.subcore_barrier()
        pltpu.sync_copy(acc.at[pl.ds(r0, RPS)],
                        out.at[pl.ds(c * N16 + r0, RPS)])

    return k


_segsum48 = _make_segsum(W1)
_segsum64 = _make_segsum(H)


def _k3_body(kacc_ref, xd_ref, z1_ref):
    accp = kacc_ref[0:N, 0:32]
    cp = kacc_ref[0:N, 32:33]
    accn = kacc_ref[N16:N16 + N, 0:32]
    cn = kacc_ref[N16:N16 + N, 32:33]
    rp = 1.0 / jnp.maximum(cp, 1.0)
    rn = 1.0 / jnp.maximum(cn, 1.0)
    pre = jnp.concatenate([accp * rp, accn * rn], axis=1) + xd_ref[...]
    z1_ref[...] = jnp.maximum(pre, 0.0)


def _k3(kacc, xd):
    return pl.pallas_call(
        _k3_body,
        out_shape=jax.ShapeDtypeStruct((N, H), jnp.float32),
    )(kacc, xd)


def _k5_body(macc_ref, kacc_ref, z1_ref, wmp_ref, wmn_ref, wz_ref, b2_ref,
             wd_ref, bd_ref, z_ref, ab_ref):
    cp = kacc_ref[0:N, 32:33]
    cn = kacc_ref[N16:N16 + N, 32:33]
    rp = 1.0 / jnp.maximum(cp, 1.0)
    rn = 1.0 / jnp.maximum(cn, 1.0)
    mp = macc_ref[0:N, :] * rp
    mn = macc_ref[N16:N16 + N, :] * rn
    z = (jnp.dot(mp, wmp_ref[...], preferred_element_type=jnp.float32)
         + jnp.dot(mn, wmn_ref[...], preferred_element_type=jnp.float32)
         + jnp.dot(z1_ref[...], wz_ref[...], preferred_element_type=jnp.float32)
         + b2_ref[...])
    z = jnp.maximum(z, 0.0)
    z_ref[...] = z
    ab_ref[...] = jnp.dot(z, wd_ref[...],
                          preferred_element_type=jnp.float32) + bd_ref[...]


def _k5(macc, kacc, z1, wmp, wmn, wz, b2, wd16, bd16):
    return pl.pallas_call(
        _k5_body,
        out_shape=[jax.ShapeDtypeStruct((N, H), jnp.float32),
                   jax.ShapeDtypeStruct((N, 16), jnp.float32)],
    )(macc, kacc, z1, wmp, wmn, wz, b2, wd16, bd16)


S_POS, S_NEG, S_NONE = 0, NW * PT6, 2 * NW * PT6
S_TOT = 2 * NW * PT6 + NW * PT6N


@functools.partial(
    pl.kernel, mesh=_mesh,
    out_type=[jax.ShapeDtypeStruct((S_TOT,), jnp.float32),
              jax.ShapeDtypeStruct((NW * 128,), jnp.float32)],
    compiler_params=pltpu.CompilerParams(use_tc_tiling_on_sc=False, needs_layout_passes=False),
    scratch_types=[
        pltpu.VMEM((CH,), jnp.int32),
        pltpu.VMEM((CH,), jnp.int32),
        pltpu.VMEM((CH,), jnp.int32),
        pltpu.VMEM((CH, 16), jnp.float32),
        pltpu.VMEM((CH, 16), jnp.float32),
        pltpu.VMEM((CH, H), jnp.float32),
        pltpu.VMEM((CH, H), jnp.float32),
        pltpu.VMEM((CH, H), jnp.float32),
        pltpu.VMEM((CH,), jnp.float32),
        pltpu.VMEM((128,), jnp.float32),
        pltpu.SemaphoreType.DMA,
    ],
)
def _k6(ab, z, pe0, pe1, ne0, ne1, no0, no1, kpp, knp,
        s_out, p_out, i0, i1, i2, bufa, bufb, bufi, bufj, bufk, sbuf,
        pbuf, sem):
    c = lax.axis_index("c")
    s = lax.axis_index("s")
    wid = c * NS + s
    iota = lax.broadcasted_iota(jnp.int32, (16,), 0)

    def nll_task(e0_hbm, e1_hbm, col, per_pad, lim, s_base):
        nch = per_pad // CH

        def chunk(ch, g_acc):
            off = wid * per_pad + ch * CH
            pltpu.sync_copy(e0_hbm.at[pl.ds(off, CH)], i0)
            pltpu.sync_copy(e1_hbm.at[pl.ds(off, CH)], i1)
            pltpu.async_copy(ab.at[i0], bufa, sem).wait()
            pltpu.async_copy(ab.at[i1], bufb, sem).wait()
            for g in range(CH // 16):
                rows = iota + g * 16
                v = []
                for j in range(3):
                    aj = plsc.load_gather(bufa, [rows, jnp.full((16,), j, jnp.int32)])
                    bj = plsc.load_gather(bufb, [rows, jnp.full((16,), j + 4, jnp.int32)])
                    v.append(aj + bj)
                m = jnp.maximum(jnp.maximum(v[0], v[1]), v[2])
                sv = (jnp.exp(v[0] - m) + jnp.exp(v[1] - m) + jnp.exp(v[2] - m))
                gv = m - v[col]
                je = ch * CH + g * 16 + iota
                mask = je < lim
                sv = jnp.where(mask, sv, 1.0)
                gv = jnp.where(mask, gv, 0.0)
                g_acc = g_acc + gv
                sbuf[pl.ds(g * 16, 16)] = sv
            pltpu.sync_copy(sbuf, s_out.at[pl.ds(s_base + off, CH)])
            return g_acc

        return lax.fori_loop(0, nch, chunk, jnp.zeros((16,), jnp.float32))

    def trip_task(e0_hbm, e1_hbm, k_hbm, sign):
        nch = PT6 // CH

        def chunk(ch, t_acc):
            off = wid * PT6 + ch * CH
            pltpu.sync_copy(e0_hbm.at[pl.ds(off, CH)], i0)
            pltpu.sync_copy(e1_hbm.at[pl.ds(off, CH)], i1)
            pltpu.sync_copy(k_hbm.at[pl.ds(off, CH)], i2)
            pltpu.async_copy(z.at[i0], bufi, sem).wait()
            pltpu.async_copy(z.at[i1], bufj, sem).wait()
            pltpu.async_copy(z.at[i2], bufk, sem).wait()

            def group(g, t_in):
                rows = iota + g * 16
                dj = jnp.zeros((16,), jnp.float32)
                dk = jnp.zeros((16,), jnp.float32)
                for dd in range(H):
                    cols = jnp.full((16,), dd, jnp.int32)
                    zi = plsc.load_gather(bufi, [rows, cols])
                    zj = plsc.load_gather(bufj, [rows, cols])
                    zk = plsc.load_gather(bufk, [rows, cols])
                    t1 = zi - zj
                    t2 = zi - zk
                    dj = dj + t1 * t1
                    dk = dk + t2 * t2
                out = (dj - dk) if sign > 0 else (dk - dj)
                out = jnp.maximum(out, 0.0)
                je = ch * CH + g * 16 + iota
                out = jnp.where(je < E // NW, out, 0.0)
                return t_in + out

            return lax.fori_loop(0, CH // 16, group, t_acc)

        return lax.fori_loop(0, nch, chunk, jnp.zeros((16,), jnp.float32))

    gp = nll_task(pe0, pe1, 0, PT6, 2500, S_POS)
    gn = nll_task(ne0, ne1, 1, PT6, 2500, S_NEG)
    g0 = nll_task(no0, no1, 2, PT6N, 5000, S_NONE)
    tp = trip_task(pe0, pe1, kpp, +1)
    tn = trip_task(ne0, ne1, knp, -1)

    pbuf[pl.ds(0, 16)] = gp
    pbuf[pl.ds(16, 16)] = gn
    pbuf[pl.ds(32, 16)] = g0
    pbuf[pl.ds(48, 16)] = tp
    pbuf[pl.ds(64, 16)] = tn
    zero16 = jnp.zeros((16,), jnp.float32)
    pbuf[pl.ds(80, 16)] = zero16
    pbuf[pl.ds(96, 16)] = zero16
    pbuf[pl.ds(112, 16)] = zero16
    pltpu.sync_copy(pbuf, p_out.at[pl.ds(wid * 128, 128)])


def _k7_body(s_ref, p_ref, out_ref):
    ls = jnp.log(s_ref[...])
    rp = NW * PT6 // 128
    slp = jnp.sum(ls[0:rp])
    sln = jnp.sum(ls[rp:2 * rp])
    sl0 = jnp.sum(ls[2 * rp:])
    q = jnp.sum(p_ref[...], axis=1)
    gp, gn, g0, tp, tn = q[0], q[1], q[2], q[3], q[4]
    fe = jnp.float32(E)
    nll = ((gp + slp) / fe + (gn + sln) / fe + (g0 + sl0) / (2 * fe)) / 3.0
    loss = nll + LAMB * (tp / fe + tn / fe)
    out_ref[0, 0] = loss


def _k7(s_flat, p_flat):
    s2 = s_flat.reshape(S_TOT // 128, 128)
    p2 = p_flat.reshape(NW, 8, 16).transpose(1, 0, 2).reshape(8, NW * 16)
    return pl.pallas_call(
        _k7_body,
        out_shape=jax.ShapeDtypeStruct((1, 1), jnp.float32),
        out_specs=pl.BlockSpec(memory_space=pltpu.SMEM),
    )(s2, p2)


def kernel(positive_edges, negative_edges, target, X,
           Wp1, bp1, Wn1, bn1, Wp2, bp2, Wn2, bn2, Wd, bd,
           none_edges, k_pos, k_neg):
    del target
    pe0, pe1 = positive_edges[0], positive_edges[1]
    ne0, ne1 = negative_edges[0], negative_edges[1]

    w1cat = jnp.concatenate([Wp1[:D], Wn1[:D], Wp1[D:], Wn1[D:]], axis=1)
    b1cat = jnp.concatenate([bp1, bn1]).reshape(1, H)
    zblk = jnp.zeros((H2, H2), jnp.float32)
    wmp = jnp.block([[Wp2[0:H2], zblk], [zblk, Wn2[0:H2]]])
    wmn = jnp.block([[zblk, Wn2[H2:2 * H2]], [Wp2[H2:2 * H2], zblk]])
    wz = jnp.block([[Wp2[2 * H2:3 * H2], zblk], [zblk, Wn2[2 * H2:3 * H2]]])
    b2 = jnp.concatenate([bp2, bn2]).reshape(1, H)
    wd16 = jnp.zeros((H, 16), jnp.float32).at[:, 0:3].set(Wd[:H]).at[:, 4:7].set(Wd[H:])
    bd16 = jnp.zeros((16,), jnp.float32).at[0:3].set(bd).reshape(1, 16)

    per2 = E // NS
    src2 = jnp.concatenate([_pad_idx(pe0, per2, PT2, 0),
                            _pad_idx(ne0, per2, PT2, 0) + N16])
    src4 = jnp.concatenate([_pad_idx(pe0, per2, PT2, 0),
                            _pad_idx(ne0, per2, PT2, 0)])
    dst24 = jnp.concatenate([_pad_idx(pe1, per2, PT2, N),
                             _pad_idx(ne1, per2, PT2, N)])
    per6 = E // NW
    pe0p = _pad_idx(pe0, per6, PT6, 0)
    pe1p = _pad_idx(pe1, per6, PT6, 0)
    ne0p = _pad_idx(ne0, per6, PT6, 0)
    ne1p = _pad_idx(ne1, per6, PT6, 0)
    no0p = _pad_idx(none_edges[0], 2 * per6, PT6N, 0)
    no1p = _pad_idx(none_edges[1], 2 * per6, PT6N, 0)
    kpp = _pad_idx(k_pos, per6, PT6, 0)
    knp = _pad_idx(k_neg, per6, PT6, 0)

    z48 = jnp.zeros((N16, W1), jnp.float32)
    z64 = jnp.zeros((N16, H), jnp.float32)

    t1, xd = _k1(X, w1cat, b1cat)
    kacc = _segsum48(t1, src2, dst24, z48)
    z1 = _k3(kacc, xd)
    macc = _segsum64(z1, src4, dst24, z64)
    z, ab = _k5(macc, kacc, z1, wmp, wmn, wz, b2, wd16, bd16)
    s_flat, p_flat = _k6(ab, z, pe0p, pe1p, ne0p, ne1p, no0p, no1p, kpp, knp)
    loss = _k7(s_flat, p_flat)[0, 0]
    return (loss, z)
